# initial kernel scaffold (unmeasured)
import jax
import jax.numpy as jnp
from jax import lax
from jax.experimental import pallas as pl
from jax.experimental.pallas import tpu as pltpu

N_DEV = 4
M = 8192
D = 2048
C = M // N_DEV


def kernel(partial, resid, gamma):
    partial2d = partial.reshape(M, D)
    gamma2d = gamma.reshape(1, D)

    def body(partial_ref, resid_ref, gamma_ref, out_ref,
             comm, stage, send_sems, recv_sems, local_sem, out_sem,
             credit_sem):
        my = lax.axis_index("i")
        right = lax.rem(my + 1, N_DEV)
        left = lax.rem(my + N_DEV - 1, N_DEV)

        barrier_sem = pltpu.get_barrier_semaphore()
        for nbr in (left, right):
            pl.semaphore_signal(
                barrier_sem, inc=1,
                device_id=(nbr,), device_id_type=pl.DeviceIdType.MESH,
            )
        pl.semaphore_wait(barrier_sem, 2)

        def load_partial_chunk(idx, dst):
            cp = pltpu.make_async_copy(
                partial_ref.at[pl.ds(idx * C, C), :], dst, local_sem)
            cp.start()
            return cp

        load_partial_chunk(my, comm.at[0]).wait()

        for t in range(2 * (N_DEV - 1)):
            send_slot = t % 2
            recv_slot = (t + 1) % 2
            if t >= 1:
                pl.semaphore_wait(credit_sem, 1)
            rdma = pltpu.make_async_remote_copy(
                src_ref=comm.at[send_slot],
                dst_ref=comm.at[recv_slot],
                send_sem=send_sems.at[send_slot],
                recv_sem=recv_sems.at[recv_slot],
                device_id=(right,),
                device_id_type=pl.DeviceIdType.MESH,
            )
            rdma.start()

            if t < N_DEV - 1:
                idx_recv = lax.rem(my + N_DEV - 1 - t, N_DEV)
                cp = load_partial_chunk(idx_recv, stage)
                cp.wait()
                rdma.wait()
                comm[recv_slot, :, :] = comm[recv_slot, :, :] + stage[:, :]
                if t == N_DEV - 2:
                    o = lax.rem(my + 1, N_DEV)
                    cpr = pltpu.make_async_copy(
                        resid_ref.at[pl.ds(o * C, C), :], stage, local_sem)
                    cpr.start()
                    cpr.wait()
                    y = comm[recv_slot, :, :] + stage[:, :]
                    rms = jnp.sqrt(
                        jnp.mean(y * y, axis=-1, keepdims=True) + 1e-6)
                    comm[recv_slot, :, :] = y / rms * gamma_ref[:, :]
                    cpo = pltpu.make_async_copy(
                        comm.at[recv_slot],
                        out_ref.at[pl.ds(o * C, C), :], out_sem)
                    cpo.start()
                    cpo.wait()
            else:
                rdma.wait()
                h = t - (N_DEV - 1)
                idx = lax.rem(my + N_DEV - h, N_DEV)
                cpo = pltpu.make_async_copy(
                    comm.at[recv_slot],
                    out_ref.at[pl.ds(idx * C, C), :], out_sem)
                cpo.start()
                cpo.wait()

            if t < 2 * (N_DEV - 1) - 1:
                pl.semaphore_signal(
                    credit_sem, inc=1,
                    device_id=(left,), device_id_type=pl.DeviceIdType.MESH,
                )

    return pl.pallas_call(
        body,
        out_shape=jax.ShapeDtypeStruct((M, D), jnp.float32),
        in_specs=[
            pl.BlockSpec(memory_space=pltpu.MemorySpace.ANY),
            pl.BlockSpec(memory_space=pltpu.MemorySpace.ANY),
            pl.BlockSpec(memory_space=pltpu.VMEM),
        ],
        out_specs=pl.BlockSpec(memory_space=pltpu.MemorySpace.ANY),
        scratch_shapes=[
            pltpu.VMEM((2, C, D), jnp.float32),
            pltpu.VMEM((C, D), jnp.float32),
            pltpu.SemaphoreType.DMA((2,)),
            pltpu.SemaphoreType.DMA((2,)),
            pltpu.SemaphoreType.DMA,
            pltpu.SemaphoreType.DMA,
            pltpu.SemaphoreType.REGULAR,
        ],
        compiler_params=pltpu.CompilerParams(collective_id=0),
    )(partial2d, resid, gamma2d)


# baseline (device time: 1202488 ns/iter reference)
import jax
import jax.numpy as jnp
from jax import lax
from jax.experimental import pallas as pl
from jax.experimental.pallas import tpu as pltpu

N_DEV = 4
M = 8192
D = 2048
C = M // N_DEV
S = 2
R = C // S
BLK = 256


def kernel(partial, resid, gamma):
    partial2d = partial.reshape(M, D)
    gamma2d = gamma.reshape(1, D)

    def body(partial_ref, resid_ref, gamma_ref, out_ref,
             comm, stage, send_sems, recv_sems, local_sem, out_sem,
             credit_sem):
        my = lax.axis_index("i")
        right = lax.rem(my + 1, N_DEV)
        left = lax.rem(my + N_DEV - 1, N_DEV)

        barrier_sem = pltpu.get_barrier_semaphore()
        for nbr in (left, right):
            pl.semaphore_signal(
                barrier_sem, inc=1,
                device_id=(nbr,), device_id_type=pl.DeviceIdType.MESH,
            )
        pl.semaphore_wait(barrier_sem, 2)

        def load(src_ref, idx, sub, dst):
            cp = pltpu.make_async_copy(
                src_ref.at[pl.ds(idx * C + sub * R, R), :], dst, local_sem)
            cp.start()
            return cp

        NSTEP = 2 * (N_DEV - 1)
        for sub in range(S):
            load(partial_ref, my, sub, comm.at[0]).wait()

            for t in range(NSTEP):
                g = sub * NSTEP + t
                send_slot = g % 2
                recv_slot = (g + 1) % 2
                if g >= 1:
                    pl.semaphore_wait(credit_sem, 1)
                rdma = pltpu.make_async_remote_copy(
                    src_ref=comm.at[send_slot],
                    dst_ref=comm.at[recv_slot],
                    send_sem=send_sems.at[send_slot],
                    recv_sem=recv_sems.at[recv_slot],
                    device_id=(right,),
                    device_id_type=pl.DeviceIdType.MESH,
                )
                rdma.start()

                if t < N_DEV - 1:
                    idx_recv = lax.rem(my + N_DEV - 1 - t, N_DEV)
                    cp = load(partial_ref, idx_recv, sub, stage)
                    cp.wait()
                    rdma.wait()
                    for b in range(R // BLK):
                        rows = pl.ds(b * BLK, BLK)
                        comm[recv_slot, rows, :] = (
                            comm[recv_slot, rows, :] + stage[rows, :])
                    if t == N_DEV - 2:
                        o = lax.rem(my + 1, N_DEV)
                        load(resid_ref, o, sub, stage).wait()
                        for b in range(R // BLK):
                            rows = pl.ds(b * BLK, BLK)
                            y = (comm[recv_slot, rows, :]
                                 + stage[rows, :])
                            rms = jnp.sqrt(
                                jnp.mean(y * y, axis=-1, keepdims=True)
                                + 1e-6)
                            comm[recv_slot, rows, :] = (
                                y / rms * gamma_ref[:, :])
                        cpo = pltpu.make_async_copy(
                            comm.at[recv_slot],
                            out_ref.at[pl.ds(o * C + sub * R, R), :],
                            out_sem)
                        cpo.start()
                        cpo.wait()
                else:
                    rdma.wait()
                    h = t - (N_DEV - 1)
                    idx = lax.rem(my + N_DEV - h, N_DEV)
                    cpo = pltpu.make_async_copy(
                        comm.at[recv_slot],
                        out_ref.at[pl.ds(idx * C + sub * R, R), :],
                        out_sem)
                    cpo.start()
                    cpo.wait()

                if g < S * NSTEP - 1:
                    pl.semaphore_signal(
                        credit_sem, inc=1,
                        device_id=(left,),
                        device_id_type=pl.DeviceIdType.MESH,
                    )

    return pl.pallas_call(
        body,
        out_shape=jax.ShapeDtypeStruct((M, D), jnp.float32),
        in_specs=[
            pl.BlockSpec(memory_space=pl.ANY),
            pl.BlockSpec(memory_space=pl.ANY),
            pl.BlockSpec(memory_space=pltpu.VMEM),
        ],
        out_specs=pl.BlockSpec(memory_space=pl.ANY),
        scratch_shapes=[
            pltpu.VMEM((2, R, D), jnp.float32),
            pltpu.VMEM((R, D), jnp.float32),
            pltpu.SemaphoreType.DMA((2,)),
            pltpu.SemaphoreType.DMA((2,)),
            pltpu.SemaphoreType.DMA,
            pltpu.SemaphoreType.DMA,
            pltpu.SemaphoreType.REGULAR,
        ],
        compiler_params=pltpu.CompilerParams(
            collective_id=0,
            vmem_limit_bytes=60 * 1024 * 1024,
        ),
    )(partial2d, resid, gamma2d)


# device time: 639570 ns/iter; 1.8802x vs baseline; 1.8802x over previous
import jax
import jax.numpy as jnp
from jax import lax
from jax.experimental import pallas as pl
from jax.experimental.pallas import tpu as pltpu

N_DEV = 4
M = 8192
D = 2048
C = M // N_DEV
H = C // 2
SUB = 2
R = H // SUB
BLK = 256
NSTEP = 2 * (N_DEV - 1)


def kernel(partial, resid, gamma):
    partial2d = partial.reshape(M, D)
    gamma2d = gamma.reshape(1, D)

    def body(partial_ref, resid_ref, gamma_ref, out_ref,
             commA, commB, stageA, stageB,
             send_semsA, recv_semsA, send_semsB, recv_semsB,
             localA, localB, outA, outB, creditA, creditB):
        my = lax.axis_index("i")
        right = lax.rem(my + 1, N_DEV)
        left = lax.rem(my + N_DEV - 1, N_DEV)

        barrier_sem = pltpu.get_barrier_semaphore()
        for nbr in (left, right):
            pl.semaphore_signal(
                barrier_sem, inc=1,
                device_id=(nbr,), device_id_type=pl.DeviceIdType.MESH,
            )
        pl.semaphore_wait(barrier_sem, 2)

        def load(src_ref, idx, half, sub, dst, sem):
            cp = pltpu.make_async_copy(
                src_ref.at[pl.ds(idx * C + half * H + sub * R, R), :],
                dst, sem)
            cp.start()
            return cp

        def store_out(comm, slot, idx, half, sub, sem):
            cp = pltpu.make_async_copy(
                comm.at[slot],
                out_ref.at[pl.ds(idx * C + half * H + sub * R, R), :],
                sem)
            cp.start()
            return cp

        def accum(comm, slot, stage):
            for b in range(R // BLK):
                rows = pl.ds(b * BLK, BLK)
                comm[slot, rows, :] = comm[slot, rows, :] + stage[rows, :]

        def ln(comm, slot, stage):
            for b in range(R // BLK):
                rows = pl.ds(b * BLK, BLK)
                y = comm[slot, rows, :] + stage[rows, :]
                rms = jnp.sqrt(
                    jnp.mean(y * y, axis=-1, keepdims=True) + 1e-6)
                comm[slot, rows, :] = y / rms * gamma_ref[:, :]

        pendA = None
        pendB = None

        oA = lax.rem(my + 1, N_DEV)
        oB = lax.rem(my + N_DEV - 1, N_DEV)

        for g in range(SUB * NSTEP):
            sub = g // NSTEP
            t = g % NSTEP
            ss = g % 2
            rs = (g + 1) % 2

            if t == 0:
                if pendA is not None:
                    pendA.wait()
                    pendA = None
                if pendB is not None:
                    pendB.wait()
                    pendB = None
                cpA = load(partial_ref, my, 0, sub, commA.at[ss], localA)
                cpB = load(partial_ref, my, 1, sub, commB.at[ss], localB)
                cpA.wait()
                cpB.wait()

            if g >= 1:
                pl.semaphore_wait(creditA, 1)
                pl.semaphore_wait(creditB, 1)
            rdmaA = pltpu.make_async_remote_copy(
                src_ref=commA.at[ss], dst_ref=commA.at[rs],
                send_sem=send_semsA.at[ss], recv_sem=recv_semsA.at[rs],
                device_id=(right,), device_id_type=pl.DeviceIdType.MESH,
            )
            rdmaB = pltpu.make_async_remote_copy(
                src_ref=commB.at[ss], dst_ref=commB.at[rs],
                send_sem=send_semsB.at[ss], recv_sem=recv_semsB.at[rs],
                device_id=(left,), device_id_type=pl.DeviceIdType.MESH,
            )
            rdmaA.start()
            rdmaB.start()

            if t < N_DEV - 1:
                idxA = lax.rem(my + N_DEV - 1 - t, N_DEV)
                idxB = lax.rem(my + t + 1, N_DEV)
                cpA = load(partial_ref, idxA, 0, sub, stageA, localA)
                cpB = load(partial_ref, idxB, 1, sub, stageB, localB)
                cpA.wait()
                cpB.wait()

            rdmaA.wait()
            rdmaB.wait()

            if pendA is not None:
                pendA.wait()
                pendA = None
            if pendB is not None:
                pendB.wait()
                pendB = None
            if g < SUB * NSTEP - 1:
                pl.semaphore_signal(
                    creditA, inc=1,
                    device_id=(left,), device_id_type=pl.DeviceIdType.MESH)
                pl.semaphore_signal(
                    creditB, inc=1,
                    device_id=(right,), device_id_type=pl.DeviceIdType.MESH)

            if t < N_DEV - 1:
                accum(commA, rs, stageA)
                accum(commB, rs, stageB)
                if t == N_DEV - 2:
                    cpA = load(resid_ref, oA, 0, sub, stageA, localA)
                    cpB = load(resid_ref, oB, 1, sub, stageB, localB)
                    cpA.wait()
                    cpB.wait()
                    ln(commA, rs, stageA)
                    ln(commB, rs, stageB)
                    pendA = store_out(commA, rs, oA, 0, sub, outA)
                    pendB = store_out(commB, rs, oB, 1, sub, outB)
            else:
                h = t - (N_DEV - 1)
                idxA = lax.rem(my + N_DEV - h, N_DEV)
                idxB = lax.rem(my + h, N_DEV)
                pendA = store_out(commA, rs, idxA, 0, sub, outA)
                pendB = store_out(commB, rs, idxB, 1, sub, outB)

        pendA.wait()
        pendB.wait()

    return pl.pallas_call(
        body,
        out_shape=jax.ShapeDtypeStruct((M, D), jnp.float32),
        in_specs=[
            pl.BlockSpec(memory_space=pl.ANY),
            pl.BlockSpec(memory_space=pl.ANY),
            pl.BlockSpec(memory_space=pltpu.VMEM),
        ],
        out_specs=pl.BlockSpec(memory_space=pl.ANY),
        scratch_shapes=[
            pltpu.VMEM((2, R, D), jnp.float32),
            pltpu.VMEM((2, R, D), jnp.float32),
            pltpu.VMEM((R, D), jnp.float32),
            pltpu.VMEM((R, D), jnp.float32),
            pltpu.SemaphoreType.DMA((2,)),
            pltpu.SemaphoreType.DMA((2,)),
            pltpu.SemaphoreType.DMA((2,)),
            pltpu.SemaphoreType.DMA((2,)),
            pltpu.SemaphoreType.DMA,
            pltpu.SemaphoreType.DMA,
            pltpu.SemaphoreType.DMA,
            pltpu.SemaphoreType.DMA,
            pltpu.SemaphoreType.REGULAR,
            pltpu.SemaphoreType.REGULAR,
        ],
        compiler_params=pltpu.CompilerParams(
            collective_id=0,
            vmem_limit_bytes=60 * 1024 * 1024,
        ),
    )(partial2d, resid, gamma2d)


# device time: 592244 ns/iter; 2.0304x vs baseline; 1.0799x over previous
import jax
import jax.numpy as jnp
from jax import lax
from jax.experimental import pallas as pl
from jax.experimental.pallas import tpu as pltpu

N_DEV = 4
M = 8192
D = 2048
C = M // N_DEV
H = C // 2
LANES = 2
GENS = 2
R = H // (LANES * GENS)
NSTEP = 2 * (N_DEV - 1)
NG = GENS * NSTEP


def kernel(partial, resid, gamma):
    partial2d = partial.reshape(M, D)
    gamma2d = gamma.reshape(1, D)

    def body(partial_ref, resid_ref, gamma_ref, out_ref,
             commA, commB, stageA, stageB,
             send_semsA, recv_semsA, send_semsB, recv_semsB,
             local_semsA, local_semsB, out_semsA, out_semsB,
             creditA0, creditA1, creditB0, creditB1):
        my = lax.axis_index("i")
        right = lax.rem(my + 1, N_DEV)
        left = lax.rem(my + N_DEV - 1, N_DEV)

        credits = {("A", 0): creditA0, ("A", 1): creditA1,
                   ("B", 0): creditB0, ("B", 1): creditB1}
        comms = {"A": commA, "B": commB}
        stages = {"A": stageA, "B": stageB}
        send_sems = {"A": send_semsA, "B": send_semsB}
        recv_sems = {"A": recv_semsA, "B": recv_semsB}
        local_sems = {"A": local_semsA, "B": local_semsB}
        out_sems = {"A": out_semsA, "B": out_semsB}
        halfs = {"A": 0, "B": 1}
        send_to = {"A": right, "B": left}
        credit_to = {"A": left, "B": right}
        owned = {"A": lax.rem(my + 1, N_DEV),
                 "B": lax.rem(my + N_DEV - 1, N_DEV)}

        barrier_sem = pltpu.get_barrier_semaphore()
        for nbr in (left, right):
            pl.semaphore_signal(
                barrier_sem, inc=1,
                device_id=(nbr,), device_id_type=pl.DeviceIdType.MESH,
            )
        pl.semaphore_wait(barrier_sem, 2)

        def rowoff(idx, d, lane, v):
            return idx * C + halfs[d] * H + (lane * GENS + v) * R

        def load(src_ref, idx, d, lane, v, dst):
            cp = pltpu.make_async_copy(
                src_ref.at[pl.ds(rowoff(idx, d, lane, v), R), :],
                dst, local_sems[d].at[lane])
            cp.start()
            return cp

        def store_out(d, lane, slot, idx, v):
            cp = pltpu.make_async_copy(
                comms[d].at[lane, slot],
                out_ref.at[pl.ds(rowoff(idx, d, lane, v), R), :],
                out_sems[d].at[lane])
            cp.start()
            return cp

        inflight = {}
        staged = {}
        pend = {(d, l): None for d in "AB" for l in range(LANES)}

        def issue(lane, G):
            t = G % NSTEP
            v = G // NSTEP
            ss = G % 2
            rs = (G + 1) % 2
            if t == 0:
                cps = []
                for d in "AB":
                    if pend[(d, lane)] is not None:
                        pend[(d, lane)].wait()
                        pend[(d, lane)] = None
                    cps.append(load(partial_ref, my, d, lane, v,
                                    comms[d].at[lane, ss]))
                for cp in cps:
                    cp.wait()
            if G >= 1:
                for d in "AB":
                    pl.semaphore_wait(credits[(d, lane)], 1)
            for d in "AB":
                rdma = pltpu.make_async_remote_copy(
                    src_ref=comms[d].at[lane, ss],
                    dst_ref=comms[d].at[lane, rs],
                    send_sem=send_sems[d].at[lane, ss],
                    recv_sem=recv_sems[d].at[lane, rs],
                    device_id=(send_to[d],),
                    device_id_type=pl.DeviceIdType.MESH,
                )
                rdma.start()
                inflight[(d, lane)] = rdma
            if t < N_DEV - 1:
                idx = {"A": lax.rem(my + N_DEV - 1 - t, N_DEV),
                       "B": lax.rem(my + t + 1, N_DEV)}
                for d in "AB":
                    staged[(d, lane)] = load(
                        partial_ref, idx[d], d, lane, v, stages[d].at[lane])

        def complete(lane, G):
            t = G % NSTEP
            v = G // NSTEP
            rs = (G + 1) % 2
            for d in "AB":
                inflight[(d, lane)].wait()
            for d in "AB":
                if pend[(d, lane)] is not None:
                    pend[(d, lane)].wait()
                    pend[(d, lane)] = None
            if G < NG - 1:
                for d in "AB":
                    pl.semaphore_signal(
                        credits[(d, lane)], inc=1,
                        device_id=(credit_to[d],),
                        device_id_type=pl.DeviceIdType.MESH)
            if t < N_DEV - 1:
                for d in "AB":
                    staged[(d, lane)].wait()
                    comm, stage = comms[d], stages[d]
                    comm[lane, rs, :, :] = (
                        comm[lane, rs, :, :] + stage[lane, :, :])
                if t == N_DEV - 2:
                    cps = [load(resid_ref, owned[d], d, lane, v,
                                stages[d].at[lane]) for d in "AB"]
                    for cp in cps:
                        cp.wait()
                    for d in "AB":
                        comm, stage = comms[d], stages[d]
                        y = comm[lane, rs, :, :] + stage[lane, :, :]
                        rms = jnp.sqrt(
                            jnp.mean(y * y, axis=-1, keepdims=True)
                            + 1e-6)
                        comm[lane, rs, :, :] = y / rms * gamma_ref[:, :]
                        pend[(d, lane)] = store_out(
                            d, lane, rs, owned[d], v)
            else:
                h = t - (N_DEV - 1)
                idx = {"A": lax.rem(my + N_DEV - h, N_DEV),
                       "B": lax.rem(my + h, N_DEV)}
                for d in "AB":
                    pend[(d, lane)] = store_out(d, lane, rs, idx[d], v)

        issue(0, 0)
        issue(1, 0)
        for G in range(NG):
            for lane in range(LANES):
                complete(lane, G)
                if G < NG - 1:
                    issue(lane, G + 1)
        for d in "AB":
            for lane in range(LANES):
                pend[(d, lane)].wait()

    return pl.pallas_call(
        body,
        out_shape=jax.ShapeDtypeStruct((M, D), jnp.float32),
        in_specs=[
            pl.BlockSpec(memory_space=pl.ANY),
            pl.BlockSpec(memory_space=pl.ANY),
            pl.BlockSpec(memory_space=pltpu.VMEM),
        ],
        out_specs=pl.BlockSpec(memory_space=pl.ANY),
        scratch_shapes=[
            pltpu.VMEM((LANES, 2, R, D), jnp.float32),
            pltpu.VMEM((LANES, 2, R, D), jnp.float32),
            pltpu.VMEM((LANES, R, D), jnp.float32),
            pltpu.VMEM((LANES, R, D), jnp.float32),
            pltpu.SemaphoreType.DMA((LANES, 2)),
            pltpu.SemaphoreType.DMA((LANES, 2)),
            pltpu.SemaphoreType.DMA((LANES, 2)),
            pltpu.SemaphoreType.DMA((LANES, 2)),
            pltpu.SemaphoreType.DMA((LANES,)),
            pltpu.SemaphoreType.DMA((LANES,)),
            pltpu.SemaphoreType.DMA((LANES,)),
            pltpu.SemaphoreType.DMA((LANES,)),
            pltpu.SemaphoreType.REGULAR,
            pltpu.SemaphoreType.REGULAR,
            pltpu.SemaphoreType.REGULAR,
            pltpu.SemaphoreType.REGULAR,
        ],
        compiler_params=pltpu.CompilerParams(
            collective_id=0,
            vmem_limit_bytes=60 * 1024 * 1024,
        ),
    )(partial2d, resid, gamma2d)
